# baseline (speedup 1.0000x reference)
import functools

import jax
import jax.numpy as jnp
from jax import lax
from jax.experimental import pallas as pl
from jax.experimental.pallas import tpu as pltpu
from jax.experimental.pallas import tpu_sc as plsc

N = 10000
D_IN = 128
H = 64
E = 320000
P = 100000

NC = 2
NS = 16
NW = NC * NS
K = 128

EC = E // K
EC_FULL = EC // NW
EC_REM = EC - EC_FULL * NW
S = 3
NSUP = EC_FULL // S

RPT = 632
RPT_LAST = N - RPT * (NS - 1)

PH = 102400
PC = PH // (NW * K)
SPG = 5

_MESH = plsc.VectorSubcoreMesh(core_axis_name="c", subcore_axis_name="s")
_SC_PARAMS = pltpu.CompilerParams(use_tc_tiling_on_sc=False,
                                  needs_layout_passes=False)
_PREC = jax.lax.Precision.HIGHEST


@functools.partial(
    pl.kernel,
    out_type=(
        jax.ShapeDtypeStruct((2 * N, H), jnp.float32),
        jax.ShapeDtypeStruct((NW, N), jnp.float32),
    ),
    mesh=_MESH,
    scratch_types=[
        pltpu.VMEM((3, S, K), jnp.int32),
        pltpu.VMEM((3, S, K), jnp.int32),
        pltpu.VMEM((2, S, K, H), jnp.float32),
        pltpu.VMEM((N,), jnp.float32),
        pltpu.VMEM_SHARED((N, H), jnp.float32),
        [pltpu.SemaphoreType.DMA] * 3,
        [pltpu.SemaphoreType.DMA] * 2,
        [pltpu.SemaphoreType.DMA] * 2,
    ],
    compiler_params=_SC_PARAMS,
)
def _segment_sum(y_hbm, src_hbm, dst_hbm, zrow_hbm, zdeg_hbm,
                 parts_hbm, degp_hbm,
                 src_v, dst_v, rows_v, deg_v, acc_sh, sem_i, sem_g, sem_s):
    cid = lax.axis_index("c")
    sid = lax.axis_index("s")
    wid = sid * NC + cid

    @pl.when(sid < NS - 1)
    def _():
        pltpu.sync_copy(zrow_hbm, acc_sh.at[pl.ds(sid * RPT, RPT)])

    @pl.when(sid == NS - 1)
    def _():
        pltpu.sync_copy(zrow_hbm.at[pl.ds(0, RPT_LAST)],
                        acc_sh.at[pl.ds(sid * RPT, RPT_LAST)])

    pltpu.sync_copy(zdeg_hbm, deg_v)
    plsc.subcore_barrier()

    ones16 = jnp.full((16,), 1.0, jnp.float32)
    base_row = wid * EC_FULL

    def fire_idx(j, ib):
        pltpu.async_copy(src_hbm.at[pl.ds(base_row + j * S, S)],
                         src_v.at[ib], sem_i[ib])
        pltpu.async_copy(dst_hbm.at[pl.ds(base_row + j * S, S)],
                         dst_v.at[ib], sem_i[ib])

    def drain_idx(ib):
        pltpu.make_async_copy(src_hbm.at[pl.ds(0, S)], src_v.at[ib],
                              sem_i[ib]).wait()
        pltpu.make_async_copy(dst_hbm.at[pl.ds(0, S)], dst_v.at[ib],
                              sem_i[ib]).wait()

    def fire_g(b, ib):
        for jj in range(S):
            pltpu.async_copy(y_hbm.at[src_v.at[ib, jj]], rows_v.at[b, jj],
                             sem_g[b])

    def drain_g(b):
        for jj in range(S):
            pltpu.make_async_copy(y_hbm.at[pl.ds(0, K)], rows_v.at[b, jj],
                                  sem_g[b]).wait()

    def fire_s(b, ib):
        for jj in range(S):
            pltpu.async_copy(rows_v.at[b, jj], acc_sh.at[dst_v.at[ib, jj]],
                             sem_s[b], add=True)

    def drain_s(b):
        for jj in range(S):
            pltpu.make_async_copy(y_hbm.at[pl.ds(0, K)], rows_v.at[b, jj],
                                  sem_s[b]).wait()

    def do_deg(ib):
        for jj in range(S):
            for j16 in range(K // 16):
                plsc.addupdate_scatter(
                    deg_v, [dst_v[ib, jj, pl.ds(j16 * 16, 16)]], ones16)

    fire_idx(0, 0)
    fire_idx(1, 1)
    drain_idx(0)
    fire_g(0, 0)

    def group(g, _):
        for r in range(6):
            j = 6 * g + r
            b, ib = r % 2, r % 3
            drain_g(b)
            @pl.when(j > 0)
            def _():
                drain_s(1 - b)
            fire_idx(j + 2, (ib + 2) % 3)
            drain_idx((ib + 1) % 3)
            fire_g(1 - b, (ib + 1) % 3)
            fire_s(b, ib)
            do_deg(ib)
        return 0

    lax.fori_loop(0, (NSUP - 2) // 6, group, 0)
    drain_g(0)
    drain_s(1)
    drain_idx(1)
    fire_g(1, 1)
    fire_s(0, 0)
    do_deg(0)
    drain_g(1)
    drain_s(0)
    fire_s(1, 1)
    do_deg(1)
    drain_s(1)

    @pl.when(wid < EC_REM)
    def _():
        pltpu.sync_copy(src_hbm.at[pl.ds(EC_FULL * NW + wid, 1)],
                        src_v.at[0, pl.ds(0, 1)])
        pltpu.sync_copy(dst_hbm.at[pl.ds(EC_FULL * NW + wid, 1)],
                        dst_v.at[0, pl.ds(0, 1)])
        pltpu.async_copy(y_hbm.at[src_v.at[0, 0]], rows_v.at[0, 0],
                         sem_g[0]).wait()
        pltpu.sync_copy(rows_v.at[0, 0], acc_sh.at[dst_v.at[0, 0]], add=True)
        for j16 in range(K // 16):
            plsc.addupdate_scatter(deg_v, [dst_v[0, 0, pl.ds(j16 * 16, 16)]],
                                   ones16)

    plsc.subcore_barrier()

    out_base = cid * N + sid * RPT

    @pl.when(sid < NS - 1)
    def _():
        pltpu.sync_copy(acc_sh.at[pl.ds(sid * RPT, RPT)],
                        parts_hbm.at[pl.ds(out_base, RPT)])

    @pl.when(sid == NS - 1)
    def _():
        pltpu.sync_copy(acc_sh.at[pl.ds(sid * RPT, RPT_LAST)],
                        parts_hbm.at[pl.ds(out_base, RPT_LAST)])

    pltpu.sync_copy(deg_v, degp_hbm.at[wid])


@functools.partial(
    pl.kernel,
    out_type=jax.ShapeDtypeStruct((PH, H), jnp.float32),
    mesh=_MESH,
    scratch_types=[
        pltpu.VMEM((PC, K), jnp.int32),
        pltpu.VMEM((2, SPG, K, H), jnp.float32),
        pltpu.SemaphoreType.DMA,
        pltpu.SemaphoreType.DMA,
    ],
    compiler_params=_SC_PARAMS,
)
def _pair_gather(h2_hbm, idx_hbm, out_hbm, idx_v, rows_v, sem_g, sem_o):
    cid = lax.axis_index("c")
    sid = lax.axis_index("s")
    wid = sid * NC + cid
    base = wid * PC

    pltpu.sync_copy(idx_hbm.at[pl.ds(base, PC)], idx_v)

    def fire_gs(j, b):
        for jj in range(SPG):
            pltpu.async_copy(h2_hbm.at[idx_v.at[j * SPG + jj]],
                             rows_v.at[b, jj], sem_g)

    def drain(b, sem):
        for _ in range(SPG):
            pltpu.make_async_copy(h2_hbm.at[pl.ds(0, K)], rows_v.at[b, 0],
                                  sem).wait()

    def fire_ws(j, b):
        for jj in range(SPG):
            pltpu.async_copy(rows_v.at[b, jj],
                             out_hbm.at[pl.ds((base + j * SPG + jj) * K, K)],
                             sem_o)

    fire_gs(0, 0)
    for j in range(PC // SPG):
        b = j % 2
        drain(b, sem_g)
        if j > 0:
            drain(1 - b, sem_o)
        if j + 1 < PC // SPG:
            fire_gs(j + 1, 1 - b)
        fire_ws(j, b)
    drain((PC // SPG - 1) % 2, sem_o)


def _mm1_body(x_ref, w_ref, b_ref, y1_ref, s1_ref):
    y = jnp.dot(x_ref[...], w_ref[...], precision=_PREC,
                preferred_element_type=jnp.float32)
    y1_ref[...] = y[:, :H]
    s1_ref[...] = y[:, H:] + b_ref[...]


def _layer_mid_body(pa_ref, pb_ref, degp_ref, s1_ref, w_ref, b_ref,
                    y2_ref, s2_ref, inv_ref):
    deg = jnp.sum(degp_ref[...], axis=1, keepdims=True)
    inv = 1.0 / jnp.maximum(deg, 1.0)
    h = jnp.maximum((pa_ref[...] + pb_ref[...]) * inv + s1_ref[...], 0.0)
    y = jnp.dot(h, w_ref[...], precision=_PREC,
                preferred_element_type=jnp.float32)
    y2_ref[...] = y[:, :H]
    s2_ref[...] = y[:, H:] + b_ref[...]
    inv_ref[...] = inv


def _layer_out_body(pa_ref, pb_ref, inv_ref, s2_ref, h2_ref):
    h2_ref[...] = (pa_ref[...] + pb_ref[...]) * inv_ref[...] + s2_ref[...]


def _pair_mlp_body(pf_ref, w_ref, a1b_ref, a2r_ref, a2b_ref,
                   m1b_ref, m2r_ref, m2b_ref, out_ref):
    pf = pf_ref[...]
    y = jnp.dot(pf, w_ref[...], preferred_element_type=jnp.float32)
    t = jnp.maximum(y[:, :H] + a1b_ref[...], 0.0)
    aw = jax.nn.sigmoid(
        jnp.sum(t * a2r_ref[...], axis=1, keepdims=True) + a2b_ref[...])
    u = jnp.maximum(aw * y[:, H:] + m1b_ref[...], 0.0)
    out_ref[...] = jax.nn.sigmoid(
        jnp.sum(u * m2r_ref[...], axis=1, keepdims=True) + m2b_ref[...])


_BN = 2000
_GN = N // _BN
_BP = 2000
_GP = P // 2 // _BP


def _full_spec(shape):
    return pl.BlockSpec(shape, lambda i: (0,) * len(shape))


def kernel(x, edge_index, profile_pairs, W1_l, W1_r, b1, W2_l, W2_r, b2,
           A1, a1b, A2, a2b, M1, m1b, M2, m2b):
    src = edge_index[0].reshape(EC, K)
    dst = edge_index[1].reshape(EC, K)
    pad = jnp.zeros((PH - P,), jnp.int32)
    idx_a = jnp.concatenate(
        [profile_pairs[: P // 2].reshape(-1), pad]).reshape(PH // K, K)
    idx_b = jnp.concatenate(
        [profile_pairs[P // 2:].reshape(-1), pad]).reshape(PH // K, K)

    w1 = jnp.concatenate([W1_l, W1_r], axis=1)
    w2 = jnp.concatenate([W2_l, W2_r], axis=1)
    b1r = b1.reshape(1, H)
    b2r = b2.reshape(1, H)
    a1br = a1b.reshape(1, H)
    a2br = a2b.reshape(1, 1)
    m1br = m1b.reshape(1, H)
    m2br = m2b.reshape(1, 1)
    a2r = A2.reshape(1, H)
    m2r = M2.reshape(1, H)

    zrow = jnp.zeros((RPT, H), jnp.float32)
    zdeg = jnp.zeros((N,), jnp.float32)

    y1, s1 = pl.pallas_call(
        _mm1_body,
        grid=(_GN,),
        in_specs=[
            pl.BlockSpec((_BN, D_IN), lambda i: (i, 0)),
            _full_spec((D_IN, 2 * H)),
            _full_spec((1, H)),
        ],
        out_specs=[
            pl.BlockSpec((_BN, H), lambda i: (i, 0)),
            pl.BlockSpec((_BN, H), lambda i: (i, 0)),
        ],
        out_shape=[
            jax.ShapeDtypeStruct((N, H), jnp.float32),
            jax.ShapeDtypeStruct((N, H), jnp.float32),
        ],
    )(x, w1, b1r)

    parts1, degp = _segment_sum(y1, src, dst, zrow, zdeg)
    degp_t = degp.T

    y2, s2, inv = pl.pallas_call(
        _layer_mid_body,
        grid=(_GN,),
        in_specs=[
            pl.BlockSpec((_BN, H), lambda i: (i, 0)),
            pl.BlockSpec((_BN, H), lambda i: (i + _GN, 0)),
            pl.BlockSpec((_BN, NW), lambda i: (i, 0)),
            pl.BlockSpec((_BN, H), lambda i: (i, 0)),
            _full_spec((H, 2 * H)),
            _full_spec((1, H)),
        ],
        out_specs=[
            pl.BlockSpec((_BN, H), lambda i: (i, 0)),
            pl.BlockSpec((_BN, H), lambda i: (i, 0)),
            pl.BlockSpec((_BN, 1), lambda i: (i, 0)),
        ],
        out_shape=[
            jax.ShapeDtypeStruct((N, H), jnp.float32),
            jax.ShapeDtypeStruct((N, H), jnp.float32),
            jax.ShapeDtypeStruct((N, 1), jnp.float32),
        ],
    )(parts1, parts1, degp_t, s1, w2, b2r)

    parts2, _ = _segment_sum(y2, src, dst, zrow, zdeg)

    h2 = pl.pallas_call(
        _layer_out_body,
        grid=(_GN,),
        in_specs=[
            pl.BlockSpec((_BN, H), lambda i: (i, 0)),
            pl.BlockSpec((_BN, H), lambda i: (i + _GN, 0)),
            pl.BlockSpec((_BN, 1), lambda i: (i, 0)),
            pl.BlockSpec((_BN, H), lambda i: (i, 0)),
        ],
        out_specs=pl.BlockSpec((_BN, H), lambda i: (i, 0)),
        out_shape=jax.ShapeDtypeStruct((N, H), jnp.float32),
    )(parts2, parts2, inv, s2)

    w_am = jnp.concatenate([A1, M1], axis=1)

    def half_mlp(pg):
        pf = pg.reshape(PH // 2, 2 * H)
        return pl.pallas_call(
            _pair_mlp_body,
            grid=(_GP,),
            in_specs=[
                pl.BlockSpec((_BP, 2 * H), lambda i: (i, 0)),
                _full_spec((2 * H, 2 * H)),
                _full_spec((1, H)),
                _full_spec((1, H)),
                _full_spec((1, 1)),
                _full_spec((1, H)),
                _full_spec((1, H)),
                _full_spec((1, 1)),
            ],
            out_specs=pl.BlockSpec((_BP, 1), lambda i: (i, 0)),
            out_shape=jax.ShapeDtypeStruct((P // 2, 1), jnp.float32),
        )(pf, w_am, a1br, a2r, a2br, m1br, m2r, m2br)

    pg_a = _pair_gather(h2, idx_a)
    pg_b = _pair_gather(h2, idx_b)
    out_a = half_mlp(pg_a)
    out_b = half_mlp(pg_b)
    return jnp.concatenate([out_a, out_b], axis=0)


# speedup vs baseline: 1.1407x; 1.1407x over previous
import functools

import jax
import jax.numpy as jnp
from jax import lax
from jax.experimental import pallas as pl
from jax.experimental.pallas import tpu as pltpu
from jax.experimental.pallas import tpu_sc as plsc

N = 10000
D_IN = 128
H = 64
E = 320000
P = 100000

NC = 2
NS = 16
NW = NC * NS
K = 128

EC = E // K
EC_FULL = EC // NW
EC_REM = EC - EC_FULL * NW
S = 3
NSUP = EC_FULL // S

RPT = 632
RPT_LAST = N - RPT * (NS - 1)

PH = 200704
PC = PH // (NW * K)
SPG = 7

_MESH = plsc.VectorSubcoreMesh(core_axis_name="c", subcore_axis_name="s")
_SC_PARAMS = pltpu.CompilerParams(use_tc_tiling_on_sc=False,
                                  needs_layout_passes=False)
_PREC = jax.lax.Precision.HIGHEST


@functools.partial(
    pl.kernel,
    out_type=(
        jax.ShapeDtypeStruct((2 * N, H), jnp.float32),
        jax.ShapeDtypeStruct((NW, N), jnp.float32),
    ),
    mesh=_MESH,
    scratch_types=[
        pltpu.VMEM((3, S, K), jnp.int32),
        pltpu.VMEM((3, S, K), jnp.int32),
        pltpu.VMEM((2, S, K, H), jnp.float32),
        pltpu.VMEM((N,), jnp.float32),
        pltpu.VMEM_SHARED((N, H), jnp.float32),
        [pltpu.SemaphoreType.DMA] * 3,
        [pltpu.SemaphoreType.DMA] * 2,
        [pltpu.SemaphoreType.DMA] * 2,
    ],
    compiler_params=_SC_PARAMS,
)
def _segment_sum(y_hbm, src_hbm, dst_hbm, zrow_hbm, zdeg_hbm,
                 parts_hbm, degp_hbm,
                 src_v, dst_v, rows_v, deg_v, acc_sh, sem_i, sem_g, sem_s):
    cid = lax.axis_index("c")
    sid = lax.axis_index("s")
    wid = sid * NC + cid

    @pl.when(sid < NS - 1)
    def _():
        pltpu.sync_copy(zrow_hbm, acc_sh.at[pl.ds(sid * RPT, RPT)])

    @pl.when(sid == NS - 1)
    def _():
        pltpu.sync_copy(zrow_hbm.at[pl.ds(0, RPT_LAST)],
                        acc_sh.at[pl.ds(sid * RPT, RPT_LAST)])

    pltpu.sync_copy(zdeg_hbm, deg_v)
    plsc.subcore_barrier()

    ones16 = jnp.full((16,), 1.0, jnp.float32)
    base_row = wid * EC_FULL

    def fire_idx(j, ib):
        pltpu.async_copy(src_hbm.at[pl.ds(base_row + j * S, S)],
                         src_v.at[ib], sem_i[ib])
        pltpu.async_copy(dst_hbm.at[pl.ds(base_row + j * S, S)],
                         dst_v.at[ib], sem_i[ib])

    def drain_idx(ib):
        pltpu.make_async_copy(src_hbm.at[pl.ds(0, S)], src_v.at[ib],
                              sem_i[ib]).wait()
        pltpu.make_async_copy(dst_hbm.at[pl.ds(0, S)], dst_v.at[ib],
                              sem_i[ib]).wait()

    def fire_g(b, ib):
        for jj in range(S):
            pltpu.async_copy(y_hbm.at[src_v.at[ib, jj]], rows_v.at[b, jj],
                             sem_g[b])

    def drain_g(b):
        for jj in range(S):
            pltpu.make_async_copy(y_hbm.at[pl.ds(0, K)], rows_v.at[b, jj],
                                  sem_g[b]).wait()

    def fire_s(b, ib):
        for jj in range(S):
            pltpu.async_copy(rows_v.at[b, jj], acc_sh.at[dst_v.at[ib, jj]],
                             sem_s[b], add=True)

    def drain_s(b):
        for jj in range(S):
            pltpu.make_async_copy(y_hbm.at[pl.ds(0, K)], rows_v.at[b, jj],
                                  sem_s[b]).wait()

    def do_deg(ib):
        for jj in range(S):
            for j16 in range(K // 16):
                plsc.addupdate_scatter(
                    deg_v, [dst_v[ib, jj, pl.ds(j16 * 16, 16)]], ones16)

    fire_idx(0, 0)
    fire_idx(1, 1)
    drain_idx(0)
    fire_g(0, 0)

    def group(g, _):
        for r in range(6):
            j = 6 * g + r
            b, ib = r % 2, r % 3
            drain_g(b)
            @pl.when(j > 0)
            def _():
                drain_s(1 - b)
            fire_idx(j + 2, (ib + 2) % 3)
            drain_idx((ib + 1) % 3)
            fire_g(1 - b, (ib + 1) % 3)
            fire_s(b, ib)
            do_deg(ib)
        return 0

    lax.fori_loop(0, (NSUP - 2) // 6, group, 0)
    drain_g(0)
    drain_s(1)
    drain_idx(1)
    fire_g(1, 1)
    fire_s(0, 0)
    do_deg(0)
    drain_g(1)
    drain_s(0)
    fire_s(1, 1)
    do_deg(1)
    drain_s(1)

    @pl.when(wid < EC_REM)
    def _():
        pltpu.sync_copy(src_hbm.at[pl.ds(EC_FULL * NW + wid, 1)],
                        src_v.at[0, pl.ds(0, 1)])
        pltpu.sync_copy(dst_hbm.at[pl.ds(EC_FULL * NW + wid, 1)],
                        dst_v.at[0, pl.ds(0, 1)])
        pltpu.async_copy(y_hbm.at[src_v.at[0, 0]], rows_v.at[0, 0],
                         sem_g[0]).wait()
        pltpu.sync_copy(rows_v.at[0, 0], acc_sh.at[dst_v.at[0, 0]], add=True)
        for j16 in range(K // 16):
            plsc.addupdate_scatter(deg_v, [dst_v[0, 0, pl.ds(j16 * 16, 16)]],
                                   ones16)

    plsc.subcore_barrier()

    out_base = cid * N + sid * RPT

    @pl.when(sid < NS - 1)
    def _():
        pltpu.sync_copy(acc_sh.at[pl.ds(sid * RPT, RPT)],
                        parts_hbm.at[pl.ds(out_base, RPT)])

    @pl.when(sid == NS - 1)
    def _():
        pltpu.sync_copy(acc_sh.at[pl.ds(sid * RPT, RPT_LAST)],
                        parts_hbm.at[pl.ds(out_base, RPT_LAST)])

    pltpu.sync_copy(deg_v, degp_hbm.at[wid])


@functools.partial(
    pl.kernel,
    out_type=jax.ShapeDtypeStruct((PH, H), jnp.float32),
    mesh=_MESH,
    scratch_types=[
        pltpu.VMEM((PC, K), jnp.int32),
        pltpu.VMEM((2, SPG, K, H), jnp.float32),
        pltpu.SemaphoreType.DMA,
        pltpu.SemaphoreType.DMA,
    ],
    compiler_params=_SC_PARAMS,
)
def _pair_gather(h2_hbm, idx_hbm, out_hbm, idx_v, rows_v, sem_g, sem_o):
    cid = lax.axis_index("c")
    sid = lax.axis_index("s")
    wid = sid * NC + cid
    base = wid * PC

    pltpu.sync_copy(idx_hbm.at[pl.ds(base, PC)], idx_v)

    def fire_gs(j, b):
        for jj in range(SPG):
            pltpu.async_copy(h2_hbm.at[idx_v.at[j * SPG + jj]],
                             rows_v.at[b, jj], sem_g)

    def drain(b, sem):
        for _ in range(SPG):
            pltpu.make_async_copy(h2_hbm.at[pl.ds(0, K)], rows_v.at[b, 0],
                                  sem).wait()

    def fire_ws(j, b):
        for jj in range(SPG):
            pltpu.async_copy(rows_v.at[b, jj],
                             out_hbm.at[pl.ds((base + j * SPG + jj) * K, K)],
                             sem_o)

    fire_gs(0, 0)
    for j in range(PC // SPG):
        b = j % 2
        drain(b, sem_g)
        if j > 0:
            drain(1 - b, sem_o)
        if j + 1 < PC // SPG:
            fire_gs(j + 1, 1 - b)
        fire_ws(j, b)
    drain((PC // SPG - 1) % 2, sem_o)


def _mm1_body(x_ref, w_ref, b_ref, y1_ref, s1_ref):
    y = jnp.dot(x_ref[...], w_ref[...], precision=_PREC,
                preferred_element_type=jnp.float32)
    y1_ref[...] = y[:, :H]
    s1_ref[...] = y[:, H:] + b_ref[...]


def _layer_mid_body(pa_ref, pb_ref, degp_ref, s1_ref, w_ref, b_ref,
                    y2_ref, s2_ref, inv_ref):
    deg = jnp.sum(degp_ref[...], axis=1, keepdims=True)
    inv = 1.0 / jnp.maximum(deg, 1.0)
    h = jnp.maximum((pa_ref[...] + pb_ref[...]) * inv + s1_ref[...], 0.0)
    y = jnp.dot(h, w_ref[...], precision=_PREC,
                preferred_element_type=jnp.float32)
    y2_ref[...] = y[:, :H]
    s2_ref[...] = y[:, H:] + b_ref[...]
    inv_ref[...] = inv


def _layer_out_body(pa_ref, pb_ref, inv_ref, s2_ref, h2_ref):
    h2_ref[...] = (pa_ref[...] + pb_ref[...]) * inv_ref[...] + s2_ref[...]


def _pair_mlp_body(pf_ref, w_ref, a1b_ref, a2r_ref, a2b_ref,
                   m1b_ref, m2r_ref, m2b_ref, out_ref):
    pf = pf_ref[...]
    y = jnp.dot(pf, w_ref[...], preferred_element_type=jnp.float32)
    t = jnp.maximum(y[:, :H] + a1b_ref[...], 0.0)
    aw = jax.nn.sigmoid(
        jnp.sum(t * a2r_ref[...], axis=1, keepdims=True) + a2b_ref[...])
    u = jnp.maximum(aw * y[:, H:] + m1b_ref[...], 0.0)
    out_ref[...] = jax.nn.sigmoid(
        jnp.sum(u * m2r_ref[...], axis=1, keepdims=True) + m2b_ref[...])


_BN = 2000
_GN = N // _BN
_BP = 2000
_GP = P // _BP


def _full_spec(shape):
    return pl.BlockSpec(shape, lambda i: (0,) * len(shape))


def kernel(x, edge_index, profile_pairs, W1_l, W1_r, b1, W2_l, W2_r, b2,
           A1, a1b, A2, a2b, M1, m1b, M2, m2b):
    src = edge_index[0].reshape(EC, K)
    dst = edge_index[1].reshape(EC, K)
    idx_all = jnp.concatenate(
        [profile_pairs.reshape(-1),
         jnp.zeros((PH - 2 * P,), jnp.int32)]).reshape(PH // K, K)

    w1 = jnp.concatenate([W1_l, W1_r], axis=1)
    w2 = jnp.concatenate([W2_l, W2_r], axis=1)
    b1r = b1.reshape(1, H)
    b2r = b2.reshape(1, H)
    a1br = a1b.reshape(1, H)
    a2br = a2b.reshape(1, 1)
    m1br = m1b.reshape(1, H)
    m2br = m2b.reshape(1, 1)
    a2r = A2.reshape(1, H)
    m2r = M2.reshape(1, H)

    zrow = jnp.zeros((RPT, H), jnp.float32)
    zdeg = jnp.zeros((N,), jnp.float32)

    y1, s1 = pl.pallas_call(
        _mm1_body,
        grid=(_GN,),
        in_specs=[
            pl.BlockSpec((_BN, D_IN), lambda i: (i, 0)),
            _full_spec((D_IN, 2 * H)),
            _full_spec((1, H)),
        ],
        out_specs=[
            pl.BlockSpec((_BN, H), lambda i: (i, 0)),
            pl.BlockSpec((_BN, H), lambda i: (i, 0)),
        ],
        out_shape=[
            jax.ShapeDtypeStruct((N, H), jnp.float32),
            jax.ShapeDtypeStruct((N, H), jnp.float32),
        ],
    )(x, w1, b1r)

    parts1, degp = _segment_sum(y1, src, dst, zrow, zdeg)
    degp_t = degp.T

    y2, s2, inv = pl.pallas_call(
        _layer_mid_body,
        grid=(_GN,),
        in_specs=[
            pl.BlockSpec((_BN, H), lambda i: (i, 0)),
            pl.BlockSpec((_BN, H), lambda i: (i + _GN, 0)),
            pl.BlockSpec((_BN, NW), lambda i: (i, 0)),
            pl.BlockSpec((_BN, H), lambda i: (i, 0)),
            _full_spec((H, 2 * H)),
            _full_spec((1, H)),
        ],
        out_specs=[
            pl.BlockSpec((_BN, H), lambda i: (i, 0)),
            pl.BlockSpec((_BN, H), lambda i: (i, 0)),
            pl.BlockSpec((_BN, 1), lambda i: (i, 0)),
        ],
        out_shape=[
            jax.ShapeDtypeStruct((N, H), jnp.float32),
            jax.ShapeDtypeStruct((N, H), jnp.float32),
            jax.ShapeDtypeStruct((N, 1), jnp.float32),
        ],
    )(parts1, parts1, degp_t, s1, w2, b2r)

    parts2, _ = _segment_sum(y2, src, dst, zrow, zdeg)

    h2 = pl.pallas_call(
        _layer_out_body,
        grid=(_GN,),
        in_specs=[
            pl.BlockSpec((_BN, H), lambda i: (i, 0)),
            pl.BlockSpec((_BN, H), lambda i: (i + _GN, 0)),
            pl.BlockSpec((_BN, 1), lambda i: (i, 0)),
            pl.BlockSpec((_BN, H), lambda i: (i, 0)),
        ],
        out_specs=pl.BlockSpec((_BN, H), lambda i: (i, 0)),
        out_shape=jax.ShapeDtypeStruct((N, H), jnp.float32),
    )(parts2, parts2, inv, s2)

    w_am = jnp.concatenate([A1, M1], axis=1)

    pg = _pair_gather(h2, idx_all)
    pf = pg.reshape(PH // 2, 2 * H)
    out = pl.pallas_call(
        _pair_mlp_body,
        grid=(_GP,),
        in_specs=[
            pl.BlockSpec((_BP, 2 * H), lambda i: (i, 0)),
            _full_spec((2 * H, 2 * H)),
            _full_spec((1, H)),
            _full_spec((1, H)),
            _full_spec((1, 1)),
            _full_spec((1, H)),
            _full_spec((1, H)),
            _full_spec((1, 1)),
        ],
        out_specs=pl.BlockSpec((_BP, 1), lambda i: (i, 0)),
        out_shape=jax.ShapeDtypeStruct((P, 1), jnp.float32),
    )(pf, w_am, a1br, a2r, a2br, m1br, m2r, m2br)
    return out
